# Initial kernel scaffold; baseline (speedup 1.0000x reference)
import jax
import jax.numpy as jnp
from jax.experimental import pallas as pl


def kernel(xyz1, xyz2):
    raise NotImplementedError("write your pallas kernel here")


# speedup vs baseline: 10.7026x; 10.7026x over previous
import functools

import jax
import jax.numpy as jnp
from jax.experimental import pallas as pl
from jax.experimental.pallas import tpu as pltpu

B = 8
N = 2048
ROWS_PER_BLK = 128
NUM_BLKS = N // ROWS_PER_BLK


def _emd_body(x1x_ref, x1y_ref, x1z_ref, x2x_ref, x2y_ref, x2z_ref,
              out_ref, dist_ref, mask_ref, acc_ref):
    blk = pl.program_id(0)

    @pl.when(blk == 0)
    def _init():
        mask_ref[...] = jnp.zeros_like(mask_ref)
        acc_ref[...] = jnp.zeros_like(acc_ref)

    x1x = x1x_ref[...][:, :, None]
    x1y = x1y_ref[...][:, :, None]
    x1z = x1z_ref[...][:, :, None]
    x2x = x2x_ref[...][None, :, :]
    x2y = x2y_ref[...][None, :, :]
    x2z = x2z_ref[...][None, :, :]
    dist_ref[...] = ((x1x - x2x) ** 2 + (x1y - x2y) ** 2
                     + (x1z - x2z) ** 2)

    iota = jax.lax.broadcasted_iota(jnp.int32, (B, N), 1)

    def step(i, _):
        row = dist_ref[i]
        m = jnp.where(mask_ref[...] > 0.5, jnp.inf, row)
        v = jnp.min(m, axis=-1, keepdims=True)
        j = jnp.min(jnp.where(m == v, iota, N), axis=-1, keepdims=True)
        mask_ref[...] = jnp.where(iota == j, 1.0, mask_ref[...])
        acc_ref[...] = acc_ref[...] + v
        return 0

    jax.lax.fori_loop(0, ROWS_PER_BLK, step, 0)

    @pl.when(blk == NUM_BLKS - 1)
    def _finish():
        out_ref[...] = acc_ref[...]


@jax.jit
def kernel(xyz1, xyz2):
    x1 = [xyz1[:, :, c].T for c in range(3)]
    x2 = [xyz2[:, :, c] for c in range(3)]

    out = pl.pallas_call(
        _emd_body,
        grid=(NUM_BLKS,),
        in_specs=[
            pl.BlockSpec((ROWS_PER_BLK, B), lambda i: (i, 0)),
            pl.BlockSpec((ROWS_PER_BLK, B), lambda i: (i, 0)),
            pl.BlockSpec((ROWS_PER_BLK, B), lambda i: (i, 0)),
            pl.BlockSpec((B, N), lambda i: (0, 0)),
            pl.BlockSpec((B, N), lambda i: (0, 0)),
            pl.BlockSpec((B, N), lambda i: (0, 0)),
        ],
        out_specs=pl.BlockSpec((B, 128), lambda i: (0, 0)),
        out_shape=jax.ShapeDtypeStruct((B, 128), jnp.float32),
        scratch_shapes=[
            pltpu.VMEM((ROWS_PER_BLK, B, N), jnp.float32),
            pltpu.VMEM((B, N), jnp.float32),
            pltpu.VMEM((B, 128), jnp.float32),
        ],
    )(*x1, *x2)

    return jnp.sum(out[:, 0]) / (B * N)


# speedup vs baseline: 20.9463x; 1.9571x over previous
import functools

import jax
import jax.numpy as jnp
from jax.experimental import pallas as pl
from jax.experimental.pallas import tpu as pltpu

B = 8
N = 2048
ROWS_PER_BLK = 128
NUM_BLKS = N // ROWS_PER_BLK


def _emd_body(x1x_ref, x1y_ref, x1z_ref, x2x_ref, x2y_ref, x2z_ref,
              out_ref, dist_ref, mask_ref, acc_ref):
    blk = pl.program_id(0)

    @pl.when(blk == 0)
    def _init():
        mask_ref[...] = jnp.zeros_like(mask_ref)
        acc_ref[...] = jnp.zeros_like(acc_ref)

    x1x = x1x_ref[...][:, :, None]
    x1y = x1y_ref[...][:, :, None]
    x1z = x1z_ref[...][:, :, None]
    x2x = x2x_ref[...][None, :, :]
    x2y = x2y_ref[...][None, :, :]
    x2z = x2z_ref[...][None, :, :]
    dist_ref[...] = ((x1x - x2x) ** 2 + (x1y - x2y) ** 2
                     + (x1z - x2z) ** 2)

    iota_f = jax.lax.broadcasted_iota(jnp.int32, (B, N), 1).astype(jnp.float32)
    big = jnp.float32(1e30)

    def step(i, carry):
        maskadd, acc = carry
        row = dist_ref[i]
        m = row + maskadd
        v = jnp.min(m, axis=-1, keepdims=True)
        jf = jnp.min(jnp.where(m == v, iota_f, jnp.float32(N)),
                     axis=-1, keepdims=True)
        maskadd = jnp.where(iota_f == jf, big, maskadd)
        return maskadd, acc + v

    maskadd, acc = jax.lax.fori_loop(
        0, ROWS_PER_BLK, step, (mask_ref[...], acc_ref[...]))
    mask_ref[...] = maskadd
    acc_ref[...] = acc

    @pl.when(blk == NUM_BLKS - 1)
    def _finish():
        out_ref[...] = acc


@jax.jit
def kernel(xyz1, xyz2):
    x1 = [xyz1[:, :, c].T for c in range(3)]
    x2 = [xyz2[:, :, c] for c in range(3)]

    out = pl.pallas_call(
        _emd_body,
        grid=(NUM_BLKS,),
        in_specs=[
            pl.BlockSpec((ROWS_PER_BLK, B), lambda i: (i, 0)),
            pl.BlockSpec((ROWS_PER_BLK, B), lambda i: (i, 0)),
            pl.BlockSpec((ROWS_PER_BLK, B), lambda i: (i, 0)),
            pl.BlockSpec((B, N), lambda i: (0, 0)),
            pl.BlockSpec((B, N), lambda i: (0, 0)),
            pl.BlockSpec((B, N), lambda i: (0, 0)),
        ],
        out_specs=pl.BlockSpec((B, 128), lambda i: (0, 0)),
        out_shape=jax.ShapeDtypeStruct((B, 128), jnp.float32),
        scratch_shapes=[
            pltpu.VMEM((ROWS_PER_BLK, B, N), jnp.float32),
            pltpu.VMEM((B, N), jnp.float32),
            pltpu.VMEM((B, 128), jnp.float32),
        ],
    )(*x1, *x2)

    return jnp.sum(out[:, 0]) / (B * N)


# speedup vs baseline: 26.1912x; 1.2504x over previous
import functools

import jax
import jax.numpy as jnp
from jax.experimental import pallas as pl
from jax.experimental.pallas import tpu as pltpu

B = 8
N = 2048
ROWS_PER_BLK = 128
NUM_BLKS = N // ROWS_PER_BLK


def _emd_body(x1x_ref, x1y_ref, x1z_ref, x2x_ref, x2y_ref, x2z_ref,
              out_ref, dist_ref, mask_ref, acc_ref):
    blk = pl.program_id(0)

    @pl.when(blk == 0)
    def _init():
        mask_ref[...] = jnp.zeros_like(mask_ref)
        acc_ref[...] = jnp.zeros_like(acc_ref)

    x1x = x1x_ref[...][:, :, None]
    x1y = x1y_ref[...][:, :, None]
    x1z = x1z_ref[...][:, :, None]
    x2x = x2x_ref[...][None, :, :]
    x2y = x2y_ref[...][None, :, :]
    x2z = x2z_ref[...][None, :, :]
    dist_ref[...] = ((x1x - x2x) ** 2 + (x1y - x2y) ** 2
                     + (x1z - x2z) ** 2)

    iota_f = jax.lax.broadcasted_iota(jnp.int32, (B, N), 1).astype(jnp.float32)
    big = jnp.float32(1e30)

    fN = jnp.float32(N)
    G = 4

    def onehot(j):
        return iota_f == j

    def exact_step(i, maskadd):
        m = dist_ref[i] + maskadd
        v = jnp.min(m, axis=-1, keepdims=True)
        j = jnp.min(jnp.where(m == v, iota_f, fN), axis=-1, keepdims=True)
        return v, j

    def group_step(p, carry):
        maskadd, acc = carry
        i0 = p * G
        vs, js = [], []
        for k in range(G):
            m = dist_ref[i0 + k] + maskadd
            vs.append(jnp.min(m, axis=-1, keepdims=True))
        for k in range(G):
            m = dist_ref[i0 + k] + maskadd
            js.append(jnp.min(jnp.where(m == vs[k], iota_f, fN),
                              axis=-1, keepdims=True))
        coll = jnp.zeros_like(js[0], dtype=bool)
        for a in range(G):
            for b_ in range(a + 1, G):
                coll = coll | (js[a] == js[b_])
        any_coll = jnp.any(coll)

        def slow(_):
            mk = jnp.where(onehot(js[0]), big, maskadd)
            svs, sjs = [vs[0]], [js[0]]
            for k in range(1, G):
                v, j = exact_step(i0 + k, mk)
                mk = jnp.where(onehot(j), big, mk)
                svs.append(v)
                sjs.append(j)
            return tuple(svs) + tuple(sjs)

        def fast(_):
            return tuple(vs) + tuple(js)

        res = jax.lax.cond(any_coll, slow, fast, 0)
        vs, js = res[:G], res[G:]
        hit = onehot(js[0])
        for k in range(1, G):
            hit = hit | onehot(js[k])
        maskadd = jnp.where(hit, big, maskadd)
        vsum = vs[0]
        for k in range(1, G):
            vsum = vsum + vs[k]
        return maskadd, acc + vsum

    maskadd, acc = jax.lax.fori_loop(
        0, ROWS_PER_BLK // G, group_step, (mask_ref[...], acc_ref[...]))
    mask_ref[...] = maskadd
    acc_ref[...] = acc

    @pl.when(blk == NUM_BLKS - 1)
    def _finish():
        out_ref[...] = acc


@jax.jit
def kernel(xyz1, xyz2):
    x1 = [xyz1[:, :, c].T for c in range(3)]
    x2 = [xyz2[:, :, c] for c in range(3)]

    out = pl.pallas_call(
        _emd_body,
        grid=(NUM_BLKS,),
        in_specs=[
            pl.BlockSpec((ROWS_PER_BLK, B), lambda i: (i, 0)),
            pl.BlockSpec((ROWS_PER_BLK, B), lambda i: (i, 0)),
            pl.BlockSpec((ROWS_PER_BLK, B), lambda i: (i, 0)),
            pl.BlockSpec((B, N), lambda i: (0, 0)),
            pl.BlockSpec((B, N), lambda i: (0, 0)),
            pl.BlockSpec((B, N), lambda i: (0, 0)),
        ],
        out_specs=pl.BlockSpec((B, 128), lambda i: (0, 0)),
        out_shape=jax.ShapeDtypeStruct((B, 128), jnp.float32),
        scratch_shapes=[
            pltpu.VMEM((ROWS_PER_BLK, B, N), jnp.float32),
            pltpu.VMEM((B, N), jnp.float32),
            pltpu.VMEM((B, 128), jnp.float32),
        ],
    )(*x1, *x2)

    return jnp.sum(out[:, 0]) / (B * N)


# speedup vs baseline: 27.4640x; 1.0486x over previous
import jax
import jax.numpy as jnp
from jax.experimental import pallas as pl
from jax.experimental.pallas import tpu as pltpu

B = 8
N = 2048
G = 4


def _emd_body(x1x_ref, x1y_ref, x1z_ref, x2x_ref, x2y_ref, x2z_ref,
              out_ref, bufa_ref, bufb_ref):
    iota_f = jax.lax.broadcasted_iota(jnp.int32, (B, N), 1).astype(jnp.float32)
    big = jnp.float32(1e30)
    fN = jnp.float32(N)

    x2x = x2x_ref[...][None, :, :]
    x2y = x2y_ref[...][None, :, :]
    x2z = x2z_ref[...][None, :, :]

    def dist8(i, buf_ref):
        x1x = x1x_ref[pl.ds(i, 8), :][:, :, None]
        x1y = x1y_ref[pl.ds(i, 8), :][:, :, None]
        x1z = x1z_ref[pl.ds(i, 8), :][:, :, None]
        buf_ref[...] = ((x1x - x2x) ** 2 + (x1y - x2y) ** 2
                        + (x1z - x2z) ** 2)

    def onehot(j):
        return iota_f == j

    def group(buf_ref, base, carry):
        maskadd, acc = carry
        vs, js = [], []
        for k in range(G):
            m = buf_ref[base + k] + maskadd
            vs.append(jnp.min(m, axis=-1, keepdims=True))
        for k in range(G):
            m = buf_ref[base + k] + maskadd
            js.append(jnp.min(jnp.where(m == vs[k], iota_f, fN),
                              axis=-1, keepdims=True))
        coll = jnp.zeros_like(js[0], dtype=bool)
        for a in range(G):
            for b_ in range(a + 1, G):
                coll = coll | (js[a] == js[b_])
        any_coll = jnp.any(coll)

        def slow(_):
            mk = jnp.where(onehot(js[0]), big, maskadd)
            svs, sjs = [vs[0]], [js[0]]
            for k in range(1, G):
                m = buf_ref[base + k] + mk
                v = jnp.min(m, axis=-1, keepdims=True)
                j = jnp.min(jnp.where(m == v, iota_f, fN),
                            axis=-1, keepdims=True)
                mk = jnp.where(onehot(j), big, mk)
                svs.append(v)
                sjs.append(j)
            return tuple(svs) + tuple(sjs)

        def fast(_):
            return tuple(vs) + tuple(js)

        res = jax.lax.cond(any_coll, slow, fast, 0)
        rvs, rjs = res[:G], res[G:]
        hit = onehot(rjs[0])
        vsum = rvs[0]
        for k in range(1, G):
            hit = hit | onehot(rjs[k])
            vsum = vsum + rvs[k]
        return jnp.where(hit, big, maskadd), acc + vsum

    dist8(0, bufa_ref)

    def iter16(k, carry):
        r0 = k * 16
        dist8(r0 + 8, bufb_ref)
        carry = group(bufa_ref, 0, carry)
        carry = group(bufa_ref, 4, carry)
        dist8(jnp.minimum(r0 + 16, N - 8), bufa_ref)
        carry = group(bufb_ref, 0, carry)
        carry = group(bufb_ref, 4, carry)
        return carry

    init = (jnp.zeros((B, N), jnp.float32), jnp.zeros((B, 128), jnp.float32))
    _, acc = jax.lax.fori_loop(0, N // 16, iter16, init)
    out_ref[...] = acc


@jax.jit
def kernel(xyz1, xyz2):
    x1 = [xyz1[:, :, c].T for c in range(3)]
    x2 = [xyz2[:, :, c] for c in range(3)]

    out = pl.pallas_call(
        _emd_body,
        out_shape=jax.ShapeDtypeStruct((B, 128), jnp.float32),
        scratch_shapes=[
            pltpu.VMEM((8, B, N), jnp.float32),
            pltpu.VMEM((8, B, N), jnp.float32),
        ],
    )(*x1, *x2)

    return jnp.sum(out[:, 0]) / (B * N)
